# baseline (speedup 1.0000x reference)
import functools

import jax
import jax.numpy as jnp
from jax import lax
from jax.experimental import pallas as pl
from jax.experimental.pallas import tpu as pltpu
from jax.experimental.pallas import tpu_sc as plsc

_EMB_D = 128
_CHUNK = 128
_NBUF = 4
_NS = 128


def _sc_gather(weight, idx2d):
    n_rows, _ = idx2d.shape
    info = plsc.get_sparse_core_info()
    nsub = info.num_subcores
    nw = info.num_cores * nsub
    nc = n_rows // nw
    ns = _NS
    nd = nc - ns
    assert ns % _NBUF == 0 and 0 < nd <= ns
    mesh = plsc.VectorSubcoreMesh(core_axis_name="c", subcore_axis_name="s")

    @functools.partial(
        pl.kernel,
        mesh=mesh,
        out_type=jax.ShapeDtypeStruct((n_rows * _CHUNK, _EMB_D), jnp.float32),
        scratch_types=(
            [pltpu.VMEM((nc, _CHUNK), jnp.int32)]
            + [pltpu.SMEM((_CHUNK,), jnp.int32)]
            + [pltpu.VMEM_SHARED((nsub, nd, _CHUNK), jnp.int32)]
            + [pltpu.VMEM((_CHUNK, _EMB_D), jnp.float32)] * _NBUF
            + [pltpu.SemaphoreType.DMA] * (2 * _NBUF + 1)
        ),
    )
    def k(table_hbm, idx_hbm, out_hbm, idx_v, idx_s, idx_sh, *bufs_and_sems):
        rows = bufs_and_sems[:_NBUF]
        gsem = bufs_and_sems[_NBUF:2 * _NBUF]
        wsem = bufs_and_sems[2 * _NBUF:3 * _NBUF]
        dsem = bufs_and_sems[3 * _NBUF]
        sid = lax.axis_index("s")
        wid = sid * info.num_cores + lax.axis_index("c")
        pltpu.sync_copy(idx_hbm.at[pl.ds(wid * nc, nc)], idx_v)
        pltpu.sync_copy(idx_v.at[pl.ds(ns, nd)], idx_sh.at[sid])

        def gather(b, j):
            pltpu.make_async_copy(
                table_hbm.at[idx_v.at[j]], rows[b], gsem[b]).start()

        for b in range(_NBUF):
            gather(b, b)

        def outer(k_, carry):
            for b in range(_NBUF):
                j = k_ * _NBUF + b
                pltpu.make_async_copy(
                    table_hbm.at[idx_v.at[j]], rows[b], gsem[b]).wait()
                base = pl.multiple_of((wid * nc + j) * _CHUNK, _CHUNK)
                cp = pltpu.make_async_copy(
                    rows[b], out_hbm.at[pl.ds(base, _CHUNK)], wsem[b])
                cp.start()

                @pl.when(j < nd)
                def _():
                    pltpu.sync_copy(idx_sh.at[sid, j], idx_s)
                    dbase = (wid * nc + ns + j) * _CHUNK

                    def row(i, c):
                        v = idx_s[i]
                        pltpu.make_async_copy(
                            table_hbm.at[pl.ds(v, 1)],
                            out_hbm.at[pl.ds(dbase + i, 1)], dsem).start()
                        return c

                    lax.fori_loop(0, _CHUNK, row, 0)

                    def drain(i, c):
                        pltpu.make_async_copy(
                            table_hbm.at[pl.ds(0, 1)],
                            out_hbm.at[pl.ds(dbase + i, 1)], dsem).wait()
                        return c

                    lax.fori_loop(0, _CHUNK, drain, 0)

                cp.wait()

                @pl.when(j + _NBUF < ns)
                def _():
                    gather(b, j + _NBUF)
            return carry

        lax.fori_loop(0, ns // _NBUF, outer, 0)

    return k(weight, idx2d)


def kernel(x, weight):
    flat = x.reshape(-1).astype(jnp.int32)
    idx2d = flat.reshape(-1, _CHUNK)
    out = _sc_gather(weight, idx2d)
    return out.reshape(x.shape + (_EMB_D,))


# speedup vs baseline: 14.2999x; 14.2999x over previous
import functools

import jax
import jax.numpy as jnp
from jax import lax
from jax.experimental import pallas as pl
from jax.experimental.pallas import tpu as pltpu
from jax.experimental.pallas import tpu_sc as plsc

_EMB_D = 128
_CHUNK = 128
_NBUF = 2


def _sc_gather(weight, idx):
    n_rows = idx.shape[0] // _CHUNK
    info = plsc.get_sparse_core_info()
    nw = info.num_cores * info.num_subcores
    nc = n_rows // nw
    mesh = plsc.VectorSubcoreMesh(core_axis_name="c", subcore_axis_name="s")

    nc2 = nc // 2
    @functools.partial(
        pl.kernel,
        mesh=mesh,
        out_type=jax.ShapeDtypeStruct((n_rows * _CHUNK, _EMB_D), jnp.float32),
        scratch_types=(
            [pltpu.VMEM((nc * _CHUNK,), jnp.int32)]
            + [pltpu.VMEM((2 * _CHUNK, _EMB_D), jnp.float32)] * _NBUF
            + [pltpu.SemaphoreType.DMA] * (2 * _NBUF)
        ),
    )
    def k(table_hbm, idx_hbm, out_hbm, idx_v, *bufs_and_sems):
        rows = bufs_and_sems[:_NBUF]
        gsem = bufs_and_sems[_NBUF:2 * _NBUF]
        wsem = bufs_and_sems[2 * _NBUF:]
        wid = lax.axis_index("s") * info.num_cores + lax.axis_index("c")
        pltpu.sync_copy(
            idx_hbm.at[pl.ds(wid * nc * _CHUNK, nc * _CHUNK)], idx_v)

        def gather(b, g):
            pltpu.make_async_copy(
                table_hbm.at[idx_v.at[pl.ds(2 * _CHUNK * g, 2 * _CHUNK)]], rows[b],
                gsem[b]).start()

        for b in range(_NBUF):
            gather(b, b)

        def outer(k_, carry):
            for b in range(_NBUF):
                g = k_ * _NBUF + b
                pltpu.make_async_copy(
                    table_hbm.at[idx_v.at[pl.ds(2 * _CHUNK * g, 2 * _CHUNK)]],
                    rows[b], gsem[b]).wait()
                base = pl.multiple_of((wid * nc2 + g) * 2 * _CHUNK, _CHUNK)
                out_slice = out_hbm.at[pl.ds(base, 2 * _CHUNK)]
                cp = pltpu.make_async_copy(rows[b], out_slice, wsem[b])
                cp.start()
                cp.wait()

                @pl.when(g + _NBUF < nc2)
                def _():
                    gather(b, g + _NBUF)
            return carry

        lax.fori_loop(0, nc2 // _NBUF, outer, 0)

    return k(weight, idx)


def kernel(x, weight):
    flat = x.reshape(-1).astype(jnp.int32)
    out = _sc_gather(weight, flat)
    return out.reshape(x.shape + (_EMB_D,))


# speedup vs baseline: 14.3292x; 1.0020x over previous
import functools

import jax
import jax.numpy as jnp
from jax import lax
from jax.experimental import pallas as pl
from jax.experimental.pallas import tpu as pltpu
from jax.experimental.pallas import tpu_sc as plsc

_EMB_D = 128
_CHUNK = 128
_NBUF = 4


def _sc_gather(weight, idx2d):
    n_rows, _ = idx2d.shape
    info = plsc.get_sparse_core_info()
    nw = info.num_cores * info.num_subcores
    nc = n_rows // nw
    mesh = plsc.VectorSubcoreMesh(core_axis_name="c", subcore_axis_name="s")

    @functools.partial(
        pl.kernel,
        mesh=mesh,
        out_type=jax.ShapeDtypeStruct((n_rows * _CHUNK, _EMB_D), jnp.float32),
        scratch_types=(
            [pltpu.VMEM((nc, _CHUNK), jnp.int32)]
            + [pltpu.VMEM((_CHUNK, _EMB_D), jnp.float32)] * _NBUF
            + [pltpu.SemaphoreType.DMA] * (2 * _NBUF)
        ),
    )
    def k(table_hbm, idx_hbm, out_hbm, idx_v, *bufs_and_sems):
        rows = bufs_and_sems[:_NBUF]
        gsem = bufs_and_sems[_NBUF:2 * _NBUF]
        wsem = bufs_and_sems[2 * _NBUF:]
        wid = lax.axis_index("s") * info.num_cores + lax.axis_index("c")
        pltpu.sync_copy(idx_hbm.at[pl.ds(wid * nc, nc)], idx_v)

        def gather(b, g):
            pltpu.make_async_copy(
                table_hbm.at[idx_v.at[g]], rows[b], gsem[b]).start()

        for b in range(_NBUF):
            gather(b, b)

        def outer(k_, carry):
            for b in range(_NBUF):
                g = k_ * _NBUF + b
                pltpu.make_async_copy(
                    table_hbm.at[idx_v.at[g]], rows[b], gsem[b]).wait()
                base = pl.multiple_of((wid * nc + g) * _CHUNK, _CHUNK)
                out_slice = out_hbm.at[pl.ds(base, _CHUNK)]
                cp = pltpu.make_async_copy(rows[b], out_slice, wsem[b])
                cp.start()
                cp.wait()

                @pl.when(g + _NBUF < nc)
                def _():
                    gather(b, g + _NBUF)
            return carry

        lax.fori_loop(0, nc // _NBUF, outer, 0)

    return k(weight, idx2d)


def kernel(x, weight):
    flat = x.reshape(-1).astype(jnp.int32)
    idx2d = flat.reshape(-1, _CHUNK)
    out = _sc_gather(weight, idx2d)
    return out.reshape(x.shape + (_EMB_D,))
